# baseline (speedup 1.0000x reference)
import functools

import jax
import jax.numpy as jnp
from jax import lax
from jax.experimental import pallas as pl
from jax.experimental.pallas import tpu as pltpu
from jax.experimental.pallas import tpu_sc as plsc

N = 10000
C = 128
H = C // 2
NPAD = 10240
RPT = NPAD // 16
W = 128
NWIN = 80
NWORK = 32
EPAD = W * NWIN * NWORK

_mesh = plsc.VectorSubcoreMesh(core_axis_name="c", subcore_axis_name="s")
_sc_params = pltpu.CompilerParams(use_tc_tiling_on_sc=False)
_tc_params = pltpu.CompilerParams(vmem_limit_bytes=100 * 1024 * 1024)


@functools.partial(
    pl.kernel,
    out_type=jax.ShapeDtypeStruct((2 * NPAD,), jnp.float32),
    mesh=_mesh,
    compiler_params=_sc_params,
    scratch_types=[
        pltpu.VMEM((NWIN, W), jnp.int32),
        pltpu.VMEM((W,), jnp.float32),
        pltpu.VMEM_SHARED((NPAD,), jnp.float32),
    ],
)
def _sc_degree(dst_hbm, ones_hbm, zeros_hbm, out_hbm, dst_v, ones_v, deg_sh):
    c = lax.axis_index("c")
    s = lax.axis_index("s")
    r0 = pl.multiple_of(s * RPT, 128)
    pltpu.sync_copy(zeros_hbm.at[pl.ds(r0, RPT)], deg_sh.at[pl.ds(r0, RPT)])
    pltpu.sync_copy(ones_hbm, ones_v)
    pltpu.sync_copy(dst_hbm.at[s].at[pl.ds(c * NWIN, NWIN)], dst_v)
    plsc.subcore_barrier()

    def body(j, carry):
        pltpu.sync_copy(ones_v, deg_sh.at[dst_v.at[j]], add=True)
        return carry

    lax.fori_loop(0, NWIN, body, 0)
    plsc.subcore_barrier()
    o0 = pl.multiple_of(c * NPAD + r0, 128)
    pltpu.sync_copy(deg_sh.at[pl.ds(r0, RPT)], out_hbm.at[pl.ds(o0, RPT)])


NWIN_T = EPAD // (16 * W)
NBUF = 4


@functools.partial(
    pl.kernel,
    out_type=jax.ShapeDtypeStruct((2, NPAD, H), jnp.float32),
    mesh=_mesh,
    compiler_params=_sc_params,
    scratch_types=[
        pltpu.VMEM((NWIN_T, W), jnp.int32),
        pltpu.VMEM((NWIN_T, W), jnp.int32),
        [pltpu.VMEM((W, H), jnp.float32)] * NBUF,
        pltpu.VMEM_SHARED((NPAD, H), jnp.float32),
        [pltpu.SemaphoreType.DMA] * NBUF,
    ],
)
def _sc_agg(h_lo_hbm, h_hi_hbm, src_hbm, dst_hbm, zeros_hbm, out_hbm,
            src_v, dst_v, bufs, acc_sh, sems):
    c = lax.axis_index("c")
    s = lax.axis_index("s")
    r0 = pl.multiple_of(s * RPT, 128)
    pltpu.sync_copy(src_hbm.at[s], src_v)
    pltpu.sync_copy(dst_hbm.at[s], dst_v)
    pltpu.sync_copy(zeros_hbm.at[pl.ds(r0, RPT)], acc_sh.at[pl.ds(r0, RPT)])
    plsc.subcore_barrier()

    def run(h_hbm):
        def wait32k(b):
            pltpu.make_async_copy(h_hbm.at[src_v.at[0]], bufs[b],
                                  sems[b]).wait()

        def stepAB(j, b):
            wait32k(b)
            pltpu.async_copy(bufs[b], acc_sh.at[dst_v.at[j]], sems[b],
                             add=True)

        def stepCD(j2, m):
            wait32k(m)
            pltpu.async_copy(h_hbm.at[src_v.at[j2]], bufs[m], sems[m])

        pltpu.async_copy(h_hbm.at[src_v.at[0]], bufs[0], sems[0])
        pltpu.async_copy(h_hbm.at[src_v.at[1]], bufs[1], sems[1])
        stepAB(0, 0)
        pltpu.async_copy(h_hbm.at[src_v.at[2]], bufs[2], sems[2])
        stepAB(1, 1)
        pltpu.async_copy(h_hbm.at[src_v.at[3]], bufs[3], sems[3])
        stepAB(2, 2)
        stepCD(4, 0)
        stepAB(3, 3)
        stepCD(5, 1)

        def body(i, carry):
            for k in range(4):
                j = 4 * i + k
                m = (k + 2) % 4
                stepAB(j, k)

                @pl.when(j + 2 < NWIN_T)
                def _():
                    stepCD(j + 2, m)

                @pl.when(j + 2 >= NWIN_T)
                def _():
                    wait32k(m)

            return carry

        lax.fori_loop(1, NWIN_T // 4, body, 0)
        wait32k(2)
        wait32k(3)

    @pl.when(c == 0)
    def _():
        run(h_lo_hbm)

    @pl.when(c == 1)
    def _():
        run(h_hi_hbm)

    plsc.subcore_barrier()
    pltpu.sync_copy(acc_sh.at[pl.ds(r0, RPT)],
                    out_hbm.at[c].at[pl.ds(r0, RPT)])


def _pre1_body(x_ref, w1_ref, degp_ref, lo_ref, hi_ref, dinv_ref):
    deg = degp_ref[0, :] + degp_ref[1, :] + 1.0
    dinv = lax.rsqrt(deg)[:, None]
    h = jnp.dot(x_ref[...], w1_ref[...], preferred_element_type=jnp.float32)
    hs = h * dinv[0:N, :]
    lo_ref[0:N, :] = hs[:, 0:H]
    hi_ref[0:N, :] = hs[:, H:C]
    zpad = jnp.zeros((NPAD - N, H), jnp.float32)
    lo_ref[N:NPAD, :] = zpad
    hi_ref[N:NPAD, :] = zpad
    dinv_ref[...] = dinv


def _tc_pre1(x, W1, deg_p):
    return pl.pallas_call(
        _pre1_body,
        out_shape=(
            jax.ShapeDtypeStruct((NPAD, H), jnp.float32),
            jax.ShapeDtypeStruct((NPAD, H), jnp.float32),
            jax.ShapeDtypeStruct((NPAD, 1), jnp.float32),
        ),
        compiler_params=_tc_params,
    )(x, W1, deg_p)


def _mid_body(acc_ref, lo_ref, hi_ref, dinv_ref, b1_ref, gam_ref, bet_ref,
              alp_ref, w2_ref, olo_ref, ohi_ref):
    dinv = dinv_ref[...]
    agg_lo = acc_ref[0] + lo_ref[...]
    agg_hi = acc_ref[1] + hi_ref[...]
    h1_lo = (agg_lo * dinv + b1_ref[:, 0:H])[0:N, :]
    h1_hi = (agg_hi * dinv + b1_ref[:, H:C])[0:N, :]
    a = alp_ref[0, 0]

    def bn_prelu(hr, gam, bet):
        m = jnp.mean(hr, axis=0, keepdims=True)
        dlt = hr - m
        var = jnp.mean(dlt * dlt, axis=0, keepdims=True)
        bn = dlt * lax.rsqrt(var + 1e-5) * gam + bet
        return jnp.where(bn > 0, bn, a * bn)

    pr_lo = bn_prelu(h1_lo, gam_ref[:, 0:H], bet_ref[:, 0:H])
    pr_hi = bn_prelu(h1_hi, gam_ref[:, H:C], bet_ref[:, H:C])
    p = (jnp.dot(pr_lo, w2_ref[0:H, :], preferred_element_type=jnp.float32)
         + jnp.dot(pr_hi, w2_ref[H:C, :], preferred_element_type=jnp.float32))
    ps = p * dinv[0:N, :]
    olo_ref[0:N, :] = ps[:, 0:H]
    ohi_ref[0:N, :] = ps[:, H:C]
    zpad = jnp.zeros((NPAD - N, H), jnp.float32)
    olo_ref[N:NPAD, :] = zpad
    ohi_ref[N:NPAD, :] = zpad


def _tc_mid(acc1, h1lo, h1hi, dinv, b1, gamma, beta, alpha, W2):
    return pl.pallas_call(
        _mid_body,
        out_shape=(
            jax.ShapeDtypeStruct((NPAD, H), jnp.float32),
            jax.ShapeDtypeStruct((NPAD, H), jnp.float32),
        ),
        compiler_params=_tc_params,
    )(acc1, h1lo, h1hi, dinv, b1, gamma, beta, alpha, W2)


def _post_body(acc_ref, lo_ref, hi_ref, dinv_ref, b2_ref, out_ref):
    dinv = dinv_ref[0:N, :]
    agg_lo = acc_ref[0, 0:N, :] + lo_ref[0:N, :]
    agg_hi = acc_ref[1, 0:N, :] + hi_ref[0:N, :]
    out_ref[:, 0:H] = agg_lo * dinv + b2_ref[:, 0:H]
    out_ref[:, H:C] = agg_hi * dinv + b2_ref[:, H:C]


def _tc_post(acc2, h2lo, h2hi, dinv, b2):
    return pl.pallas_call(
        _post_body,
        out_shape=jax.ShapeDtypeStruct((N, C), jnp.float32),
        compiler_params=_tc_params,
    )(acc2, h2lo, h2hi, dinv, b2)


def kernel(x, edge_index, W1, b1, gamma, beta, alpha, W2, b2):
    src = edge_index[0].astype(jnp.int32)
    dst = edge_index[1].astype(jnp.int32)
    npad = EPAD - src.shape[0]
    padidx = jnp.arange(npad, dtype=jnp.int32) % 16 + N
    src3 = jnp.concatenate([src, padidx]).reshape(16, NWIN_T, W)
    dst3 = jnp.concatenate([dst, padidx]).reshape(16, NWIN_T, W)
    zeros2d = jnp.zeros((NPAD, H), jnp.float32)
    zeros1d = jnp.zeros((NPAD,), jnp.float32)
    ones1 = jnp.ones((W,), jnp.float32)

    deg_p = _sc_degree(dst3, ones1, zeros1d).reshape(2, NPAD)
    h1lo, h1hi, dinv = _tc_pre1(x, W1, deg_p)
    acc1 = _sc_agg(h1lo, h1hi, src3, dst3, zeros2d)
    h2lo, h2hi = _tc_mid(acc1, h1lo, h1hi, dinv,
                         b1.reshape(1, C), gamma.reshape(1, C),
                         beta.reshape(1, C), jnp.reshape(alpha, (1, 1)), W2)
    acc2 = _sc_agg(h2lo, h2hi, src3, dst3, zeros2d)
    return _tc_post(acc2, h2lo, h2hi, dinv, b2.reshape(1, C))


# speedup vs baseline: 1.1815x; 1.1815x over previous
import functools

import jax
import jax.numpy as jnp
from jax import lax
from jax.experimental import pallas as pl
from jax.experimental.pallas import tpu as pltpu
from jax.experimental.pallas import tpu_sc as plsc

N = 10000
C = 128
NPAD = 10240
RPT = NPAD // 16
W = 128
NWIN = 80
NWIN_T = 2 * NWIN
EPAD = W * NWIN_T * 16

_mesh = plsc.VectorSubcoreMesh(core_axis_name="c", subcore_axis_name="s")
_tc_params = pltpu.CompilerParams(vmem_limit_bytes=100 * 1024 * 1024)


@functools.partial(
    pl.kernel,
    out_type=jax.ShapeDtypeStruct((2 * NPAD,), jnp.float32),
    mesh=_mesh,
    scratch_types=[
        pltpu.VMEM((NWIN, W), jnp.int32),
        pltpu.VMEM((W,), jnp.float32),
        pltpu.VMEM_SHARED((NPAD,), jnp.float32),
    ],
)
def _sc_degree(dst_hbm, ones_hbm, zeros_hbm, out_hbm, dst_v, ones_v, deg_sh):
    c = lax.axis_index("c")
    s = lax.axis_index("s")
    r0 = pl.multiple_of(s * RPT, 128)
    pltpu.sync_copy(zeros_hbm.at[pl.ds(r0, RPT)], deg_sh.at[pl.ds(r0, RPT)])
    pltpu.sync_copy(ones_hbm, ones_v)
    pltpu.sync_copy(dst_hbm.at[s].at[pl.ds(c * NWIN, NWIN)], dst_v)
    plsc.subcore_barrier()

    def body(j, carry):
        pltpu.sync_copy(ones_v, deg_sh.at[dst_v.at[j]], add=True)
        return carry

    lax.fori_loop(0, NWIN, body, 0)
    plsc.subcore_barrier()
    o0 = pl.multiple_of(c * NPAD + r0, 128)
    pltpu.sync_copy(deg_sh.at[pl.ds(r0, RPT)], out_hbm.at[pl.ds(o0, RPT)])


CH = 16
NCH = NWIN // CH


@functools.partial(
    pl.kernel,
    out_type=jax.ShapeDtypeStruct((2, NPAD, C), jnp.float32),
    mesh=_mesh,
    scratch_types=[
        [pltpu.VMEM((CH, W), jnp.int32)] * 2,
        [pltpu.VMEM((CH, W), jnp.int32)] * 2,
        [pltpu.VMEM((W, C), jnp.float32)] * 2,
        pltpu.VMEM_SHARED((NPAD, C), jnp.float32),
        pltpu.SemaphoreType.DMA,
        pltpu.SemaphoreType.DMA,
        pltpu.SemaphoreType.DMA,
    ],
)
def _sc_agg(h_hbm, src_hbm, dst_hbm, zeros_hbm, out_hbm,
            srcc, dstc, bufs, acc_sh, gsem0, gsem1, isem):
    c = lax.axis_index("c")
    s = lax.axis_index("s")
    r0 = pl.multiple_of(s * RPT, 128)
    w0 = c * NWIN
    gsems = (gsem0, gsem1)

    def fire_chunk(k, p):
        pltpu.async_copy(src_hbm.at[s].at[pl.ds(w0 + k * CH, CH)],
                         srcc[p], isem)
        pltpu.async_copy(dst_hbm.at[s].at[pl.ds(w0 + k * CH, CH)],
                         dstc[p], isem)

    def wait_chunk(p):
        pltpu.make_async_copy(src_hbm.at[s].at[pl.ds(w0, CH)],
                              srcc[p], isem).wait()
        pltpu.make_async_copy(dst_hbm.at[s].at[pl.ds(w0, CH)],
                              dstc[p], isem).wait()

    pltpu.sync_copy(src_hbm.at[s].at[pl.ds(w0, CH)], srcc[0])
    pltpu.sync_copy(dst_hbm.at[s].at[pl.ds(w0, CH)], dstc[0])
    pltpu.sync_copy(zeros_hbm.at[pl.ds(r0, RPT)], acc_sh.at[pl.ds(r0, RPT)])
    plsc.subcore_barrier()

    def chunk_block(k):
        p = k % 2
        sv, dv = srcc[p], dstc[p]
        if k > 0:
            wait_chunk(p)
        if k + 1 < NCH:
            fire_chunk(k + 1, (k + 1) % 2)

        pltpu.async_copy(h_hbm.at[sv.at[0]], bufs[0], gsems[0])
        pltpu.async_copy(h_hbm.at[sv.at[1]], bufs[1], gsems[1])

        def wbody(i, carry):
            for l in range(2):
                j = 2 * i + l
                pltpu.make_async_copy(h_hbm.at[sv.at[j]], bufs[l],
                                      gsems[l]).wait()
                pltpu.sync_copy(bufs[l], acc_sh.at[dv.at[j]], add=True)

                @pl.when(j + 2 < CH)
                def _():
                    pltpu.async_copy(h_hbm.at[sv.at[j + 2]], bufs[l],
                                     gsems[l])

            return carry

        lax.fori_loop(0, CH // 2, wbody, 0)

    for k in range(NCH):
        chunk_block(k)

    plsc.subcore_barrier()
    pltpu.sync_copy(acc_sh.at[pl.ds(r0, RPT)],
                    out_hbm.at[c].at[pl.ds(r0, RPT)])


def _pre1_body(x_ref, w1_ref, degp_ref, hs_ref, dinv_ref):
    deg = degp_ref[0, :] + degp_ref[1, :] + 1.0
    dinv = lax.rsqrt(deg)[:, None]
    h = jnp.dot(x_ref[...], w1_ref[...], preferred_element_type=jnp.float32)
    hs_ref[0:N, :] = h * dinv[0:N, :]
    hs_ref[N:NPAD, :] = jnp.zeros((NPAD - N, C), jnp.float32)
    dinv_ref[...] = dinv


def _tc_pre1(x, W1, deg_p):
    return pl.pallas_call(
        _pre1_body,
        out_shape=(
            jax.ShapeDtypeStruct((NPAD, C), jnp.float32),
            jax.ShapeDtypeStruct((NPAD, 1), jnp.float32),
        ),
        compiler_params=_tc_params,
    )(x, W1, deg_p)


def _mid_body(acc_ref, hs_ref, dinv_ref, b1_ref, gam_ref, bet_ref, alp_ref,
              w2_ref, out_ref):
    dinv = dinv_ref[...]
    agg = acc_ref[0] + acc_ref[1] + hs_ref[...]
    hr = (agg * dinv + b1_ref[...])[0:N, :]
    m = jnp.mean(hr, axis=0, keepdims=True)
    dlt = hr - m
    var = jnp.mean(dlt * dlt, axis=0, keepdims=True)
    bn = dlt * lax.rsqrt(var + 1e-5) * gam_ref[...] + bet_ref[...]
    a = alp_ref[0, 0]
    pr = jnp.where(bn > 0, bn, a * bn)
    p = jnp.dot(pr, w2_ref[...], preferred_element_type=jnp.float32)
    out_ref[0:N, :] = p * dinv[0:N, :]
    out_ref[N:NPAD, :] = jnp.zeros((NPAD - N, C), jnp.float32)


def _tc_mid(acc1, h1s, dinv, b1, gamma, beta, alpha, W2):
    return pl.pallas_call(
        _mid_body,
        out_shape=jax.ShapeDtypeStruct((NPAD, C), jnp.float32),
        compiler_params=_tc_params,
    )(acc1, h1s, dinv, b1, gamma, beta, alpha, W2)


def _post_body(acc_ref, hs_ref, dinv_ref, b2_ref, out_ref):
    agg = acc_ref[0, 0:N, :] + acc_ref[1, 0:N, :] + hs_ref[0:N, :]
    out_ref[...] = agg * dinv_ref[0:N, :] + b2_ref[...]


def _tc_post(acc2, h2s, dinv, b2):
    return pl.pallas_call(
        _post_body,
        out_shape=jax.ShapeDtypeStruct((N, C), jnp.float32),
        compiler_params=_tc_params,
    )(acc2, h2s, dinv, b2)


def kernel(x, edge_index, W1, b1, gamma, beta, alpha, W2, b2):
    src = edge_index[0].astype(jnp.int32)
    dst = edge_index[1].astype(jnp.int32)
    npad = EPAD - src.shape[0]
    padidx = jnp.arange(npad, dtype=jnp.int32) % 16 + N
    src3 = jnp.concatenate([src, padidx]).reshape(16, NWIN_T, W)
    dst3 = jnp.concatenate([dst, padidx]).reshape(16, NWIN_T, W)
    zeros2d = jnp.zeros((NPAD, C), jnp.float32)
    zeros1d = jnp.zeros((NPAD,), jnp.float32)
    ones1 = jnp.ones((W,), jnp.float32)

    deg_p = _sc_degree(dst3, ones1, zeros1d).reshape(2, NPAD)
    h1s, dinv = _tc_pre1(x, W1, deg_p)
    acc1 = _sc_agg(h1s, src3, dst3, zeros2d)
    h2s = _tc_mid(acc1, h1s, dinv,
                  b1.reshape(1, C), gamma.reshape(1, C), beta.reshape(1, C),
                  jnp.reshape(alpha, (1, 1)), W2)
    acc2 = _sc_agg(h2s, src3, dst3, zeros2d)
    return _tc_post(acc2, h2s, dinv, b2.reshape(1, C))


# speedup vs baseline: 1.2465x; 1.0550x over previous
import functools

import jax
import jax.numpy as jnp
from jax import lax
from jax.experimental import pallas as pl
from jax.experimental.pallas import tpu as pltpu
from jax.experimental.pallas import tpu_sc as plsc

N = 10000
C = 128
NPAD = 10240
RPT = NPAD // 16
W = 128
NWIN = 80
NWIN_T = 2 * NWIN
EPAD = W * NWIN_T * 16

_mesh = plsc.VectorSubcoreMesh(core_axis_name="c", subcore_axis_name="s")
_tc_params = pltpu.CompilerParams(vmem_limit_bytes=100 * 1024 * 1024)


@functools.partial(
    pl.kernel,
    out_type=jax.ShapeDtypeStruct((2 * NPAD,), jnp.float32),
    mesh=_mesh,
    scratch_types=[
        pltpu.VMEM((NWIN, W), jnp.int32),
        pltpu.VMEM((W,), jnp.float32),
        pltpu.VMEM_SHARED((NPAD,), jnp.float32),
        pltpu.SemaphoreType.DMA,
    ],
)
def _sc_degree(dst_hbm, ones_hbm, zeros_hbm, out_hbm, dst_v, ones_v, deg_sh,
               dsem):
    c = lax.axis_index("c")
    s = lax.axis_index("s")
    r0 = pl.multiple_of(s * RPT, 128)
    pltpu.sync_copy(zeros_hbm.at[pl.ds(r0, RPT)], deg_sh.at[pl.ds(r0, RPT)])
    pltpu.sync_copy(ones_hbm, ones_v)
    pltpu.sync_copy(dst_hbm.at[s].at[pl.ds(c * NWIN, NWIN)], dst_v)
    plsc.subcore_barrier()

    def fire(j):
        pltpu.async_copy(ones_v, deg_sh.at[dst_v.at[j]], dsem, add=True)

    def drain(carry=0):
        pltpu.make_async_copy(ones_v, deg_sh.at[dst_v.at[0]], dsem).wait()

    def body(j, carry):
        fire(j)
        drain()
        return carry

    for j in range(8):
        fire(j)
    lax.fori_loop(8, NWIN, body, 0)
    lax.fori_loop(0, 8, lambda j, cy: (drain(), cy)[1], 0)
    plsc.subcore_barrier()
    o0 = pl.multiple_of(c * NPAD + r0, 128)
    pltpu.sync_copy(deg_sh.at[pl.ds(r0, RPT)], out_hbm.at[pl.ds(o0, RPT)])


CH = 16
NCH = NWIN // CH


@functools.partial(
    pl.kernel,
    out_type=jax.ShapeDtypeStruct((2, NPAD, C), jnp.float32),
    mesh=_mesh,
    scratch_types=[
        [pltpu.VMEM((CH, W), jnp.int32)] * 2,
        [pltpu.VMEM((CH, W), jnp.int32)] * 2,
        [pltpu.VMEM((W, C), jnp.float32)] * 2,
        pltpu.VMEM_SHARED((NPAD, C), jnp.float32),
        pltpu.SemaphoreType.DMA,
        pltpu.SemaphoreType.DMA,
        pltpu.SemaphoreType.DMA,
    ],
)
def _sc_agg(h_hbm, src_hbm, dst_hbm, zeros_hbm, out_hbm,
            srcc, dstc, bufs, acc_sh, gsem0, gsem1, isem):
    c = lax.axis_index("c")
    s = lax.axis_index("s")
    r0 = pl.multiple_of(s * RPT, 128)
    w0 = c * NWIN
    gsems = (gsem0, gsem1)

    def fire_chunk(k, p):
        pltpu.async_copy(src_hbm.at[s].at[pl.ds(w0 + k * CH, CH)],
                         srcc[p], isem)
        pltpu.async_copy(dst_hbm.at[s].at[pl.ds(w0 + k * CH, CH)],
                         dstc[p], isem)

    def wait_chunk(p):
        pltpu.make_async_copy(src_hbm.at[s].at[pl.ds(w0, CH)],
                              srcc[p], isem).wait()
        pltpu.make_async_copy(dst_hbm.at[s].at[pl.ds(w0, CH)],
                              dstc[p], isem).wait()

    pltpu.sync_copy(src_hbm.at[s].at[pl.ds(w0, CH)], srcc[0])
    pltpu.sync_copy(dst_hbm.at[s].at[pl.ds(w0, CH)], dstc[0])
    pltpu.sync_copy(zeros_hbm.at[pl.ds(r0, RPT)], acc_sh.at[pl.ds(r0, RPT)])
    plsc.subcore_barrier()

    def chunk_block(k):
        p = k % 2
        sv, dv = srcc[p], dstc[p]
        if k + 1 < NCH:
            fire_chunk(k + 1, (k + 1) % 2)

        def wbody(i, carry):
            for l in range(2):
                j = 2 * i + l
                pltpu.make_async_copy(h_hbm.at[sv.at[j]], bufs[l],
                                      gsems[l]).wait()
                pltpu.sync_copy(bufs[l], acc_sh.at[dv.at[j]], add=True)
                pltpu.async_copy(h_hbm.at[sv.at[j + 2]], bufs[l], gsems[l])

            return carry

        lax.fori_loop(0, CH // 2 - 1, wbody, 0)

        if k + 1 < NCH:
            wait_chunk((k + 1) % 2)
            nsv = srcc[(k + 1) % 2]
        for l in range(2):
            j = CH - 2 + l
            pltpu.make_async_copy(h_hbm.at[sv.at[j]], bufs[l],
                                  gsems[l]).wait()
            pltpu.sync_copy(bufs[l], acc_sh.at[dv.at[j]], add=True)
            if k + 1 < NCH:
                pltpu.async_copy(h_hbm.at[nsv.at[l]], bufs[l], gsems[l])

    pltpu.async_copy(h_hbm.at[srcc[0].at[0]], bufs[0], gsems[0])
    pltpu.async_copy(h_hbm.at[srcc[0].at[1]], bufs[1], gsems[1])
    for k in range(NCH):
        chunk_block(k)

    plsc.subcore_barrier()
    pltpu.sync_copy(acc_sh.at[pl.ds(r0, RPT)],
                    out_hbm.at[c].at[pl.ds(r0, RPT)])


def _pre1_body(x_ref, w1_ref, degp_ref, hs_ref, dinv_ref):
    deg = degp_ref[0, :] + degp_ref[1, :] + 1.0
    dinv = lax.rsqrt(deg)[:, None]
    h = jnp.dot(x_ref[...], w1_ref[...], preferred_element_type=jnp.float32)
    hs_ref[0:N, :] = h * dinv[0:N, :]
    hs_ref[N:NPAD, :] = jnp.zeros((NPAD - N, C), jnp.float32)
    dinv_ref[...] = dinv


def _tc_pre1(x, W1, deg_p):
    return pl.pallas_call(
        _pre1_body,
        out_shape=(
            jax.ShapeDtypeStruct((NPAD, C), jnp.float32),
            jax.ShapeDtypeStruct((NPAD, 1), jnp.float32),
        ),
        compiler_params=_tc_params,
    )(x, W1, deg_p)


def _mid_body(acc_ref, hs_ref, dinv_ref, b1_ref, gam_ref, bet_ref, alp_ref,
              w2_ref, out_ref):
    dinv = dinv_ref[...]
    agg = acc_ref[0] + acc_ref[1] + hs_ref[...]
    hr = (agg * dinv + b1_ref[...])[0:N, :]
    m = jnp.mean(hr, axis=0, keepdims=True)
    dlt = hr - m
    var = jnp.mean(dlt * dlt, axis=0, keepdims=True)
    bn = dlt * lax.rsqrt(var + 1e-5) * gam_ref[...] + bet_ref[...]
    a = alp_ref[0, 0]
    pr = jnp.where(bn > 0, bn, a * bn)
    p = jnp.dot(pr, w2_ref[...], preferred_element_type=jnp.float32)
    out_ref[0:N, :] = p * dinv[0:N, :]
    out_ref[N:NPAD, :] = jnp.zeros((NPAD - N, C), jnp.float32)


def _tc_mid(acc1, h1s, dinv, b1, gamma, beta, alpha, W2):
    return pl.pallas_call(
        _mid_body,
        out_shape=jax.ShapeDtypeStruct((NPAD, C), jnp.float32),
        compiler_params=_tc_params,
    )(acc1, h1s, dinv, b1, gamma, beta, alpha, W2)


def _post_body(acc_ref, hs_ref, dinv_ref, b2_ref, out_ref):
    agg = acc_ref[0, 0:N, :] + acc_ref[1, 0:N, :] + hs_ref[0:N, :]
    out_ref[...] = agg * dinv_ref[0:N, :] + b2_ref[...]


def _tc_post(acc2, h2s, dinv, b2):
    return pl.pallas_call(
        _post_body,
        out_shape=jax.ShapeDtypeStruct((N, C), jnp.float32),
        compiler_params=_tc_params,
    )(acc2, h2s, dinv, b2)


def kernel(x, edge_index, W1, b1, gamma, beta, alpha, W2, b2):
    src = edge_index[0].astype(jnp.int32)
    dst = edge_index[1].astype(jnp.int32)
    npad = EPAD - src.shape[0]
    padidx = jnp.arange(npad, dtype=jnp.int32) % 16 + N
    src3 = jnp.concatenate([src, padidx]).reshape(16, NWIN_T, W)
    dst3 = jnp.concatenate([dst, padidx]).reshape(16, NWIN_T, W)
    zeros2d = jnp.zeros((NPAD, C), jnp.float32)
    zeros1d = jnp.zeros((NPAD,), jnp.float32)
    ones1 = jnp.ones((W,), jnp.float32)

    deg_p = _sc_degree(dst3, ones1, zeros1d).reshape(2, NPAD)
    h1s, dinv = _tc_pre1(x, W1, deg_p)
    acc1 = _sc_agg(h1s, src3, dst3, zeros2d)
    h2s = _tc_mid(acc1, h1s, dinv,
                  b1.reshape(1, C), gamma.reshape(1, C), beta.reshape(1, C),
                  jnp.reshape(alpha, (1, 1)), W2)
    acc2 = _sc_agg(h2s, src3, dst3, zeros2d)
    return _tc_post(acc2, h2s, dinv, b2.reshape(1, C))
